# Initial kernel scaffold; baseline (speedup 1.0000x reference)
import jax
import jax.numpy as jnp
from jax.experimental import pallas as pl


def kernel(x, edge_index, edge_attr, batch, params):
    raise NotImplementedError("write your pallas kernel here")


# speedup vs baseline: 6.5624x; 6.5624x over previous
import functools

import jax
import jax.numpy as jnp
from jax import lax
from jax.experimental import pallas as pl
from jax.experimental.pallas import tpu as pltpu
from jax.experimental.pallas import tpu_sc as plsc

N = 10000
E = 320000
D = 128
EHID = 256
EDIM = 16
G = 64

NC = 2
NS = 16
NW = NC * NS
EPW = E // NW
CH = 80
NCHUNK = EPW // CH

@functools.cache
def _sc_mesh():
    return plsc.VectorSubcoreMesh(
        core_axis_name="c", subcore_axis_name="s",
        num_cores=NC, num_subcores=NS)


_f32 = jnp.float32


def _node0_body(x_ref, a_ref, b_ref, b1_ref, lw_ref, asw_ref, adw_ref,
                xa_ref, xb_ref, h_ref, as_ref, ad_ref):
    x = x_ref[...]
    xa_ref[...] = jnp.dot(x, a_ref[...], preferred_element_type=_f32) \
        + b1_ref[...][None, :]
    xb_ref[...] = jnp.dot(x, b_ref[...], preferred_element_type=_f32)
    h = jnp.dot(x, lw_ref[...], preferred_element_type=_f32)
    h_ref[...] = h
    as_ref[...] = jnp.sum(h * asw_ref[...][None, :], axis=1)
    ad_ref[...] = jnp.sum(h * adw_ref[...][None, :], axis=1)


_node0 = pl.pallas_call(
    _node0_body,
    out_shape=(
        jax.ShapeDtypeStruct((N, EHID), _f32),
        jax.ShapeDtypeStruct((N, EHID), _f32),
        jax.ShapeDtypeStruct((N, D), _f32),
        jax.ShapeDtypeStruct((N,), _f32),
        jax.ShapeDtypeStruct((N,), _f32),
    ),
)


def _node_epilogue(pp_ref, dp_ref, gb_ref, gam_ref, bet_ref):
    p = pp_ref[0] + pp_ref[1]
    den = dp_ref[0] + dp_ref[1]
    out = p / (den + 1e-16)[:, None] + gb_ref[...][None, :]
    mu = jnp.mean(out, axis=0)
    var = jnp.mean((out - mu[None, :]) ** 2, axis=0)
    xn = (out - mu[None, :]) / jnp.sqrt(var + 1e-5) \
        * gam_ref[...][None, :] + bet_ref[...][None, :]
    return jnp.maximum(xn, 0.0)


def _nodefin_body(pp_ref, dp_ref, gb_ref, gam_ref, bet_ref,
                  a_ref, b_ref, b1_ref, lw_ref, asw_ref, adw_ref,
                  xa_ref, xb_ref, h_ref, as_ref, ad_ref):
    x = _node_epilogue(pp_ref, dp_ref, gb_ref, gam_ref, bet_ref)
    xa_ref[...] = jnp.dot(x, a_ref[...], preferred_element_type=_f32) \
        + b1_ref[...][None, :]
    xb_ref[...] = jnp.dot(x, b_ref[...], preferred_element_type=_f32)
    h = jnp.dot(x, lw_ref[...], preferred_element_type=_f32)
    h_ref[...] = h
    as_ref[...] = jnp.sum(h * asw_ref[...][None, :], axis=1)
    ad_ref[...] = jnp.sum(h * adw_ref[...][None, :], axis=1)


_nodefin = pl.pallas_call(
    _nodefin_body,
    out_shape=(
        jax.ShapeDtypeStruct((N, EHID), _f32),
        jax.ShapeDtypeStruct((N, EHID), _f32),
        jax.ShapeDtypeStruct((N, D), _f32),
        jax.ShapeDtypeStruct((N,), _f32),
        jax.ShapeDtypeStruct((N,), _f32),
    ),
)


def _scgather_body(xa_hbm, xb_hbm, asrc_hbm, adst_hbm, src_hbm, dst_hbm,
                   gsum_hbm, ag_hbm, adg_hbm,
                   idx_s, idx_d, rows_a, rows_b, asrc_v, adst_v,
                   ag_v, adg_v, sem_a, sem_b):
    cid = lax.axis_index("c")
    sid = lax.axis_index("s")
    wid = sid * NC + cid
    base = wid * EPW
    pltpu.sync_copy(asrc_hbm, asrc_v)
    pltpu.sync_copy(adst_hbm, adst_v)

    def chunk(i, carry):
        off = base + i * CH
        pltpu.sync_copy(src_hbm.at[pl.ds(off, CH)], idx_s)
        pltpu.sync_copy(dst_hbm.at[pl.ds(off, CH)], idx_d)
        ca = pltpu.async_copy(xa_hbm.at[idx_s], rows_a, sem_a)
        cb = pltpu.async_copy(xb_hbm.at[idx_d], rows_b, sem_b)
        ca.wait()
        cb.wait()

        def addrow(e, c2):
            for k in range(EHID // 16):
                sl = pl.ds(k * 16, 16)
                rows_a[e, sl] = rows_a[e, sl] + rows_b[e, sl]
            return c2
        lax.fori_loop(0, CH, addrow, 0)

        for j in range(CH // 16):
            sl = pl.ds(j * 16, 16)
            ag_v[sl] = plsc.load_gather(asrc_v, [idx_s[sl]])
            adg_v[sl] = plsc.load_gather(adst_v, [idx_d[sl]])

        pltpu.sync_copy(rows_a, gsum_hbm.at[pl.ds(off, CH)])
        pltpu.sync_copy(ag_v, ag_hbm.at[pl.ds(off, CH)])
        pltpu.sync_copy(adg_v, adg_hbm.at[pl.ds(off, CH)])
        return carry
    lax.fori_loop(0, NCHUNK, chunk, 0)


@functools.cache
def _scgather():
  return pl.kernel(
    _scgather_body,
    out_type=(
        jax.ShapeDtypeStruct((E, EHID), _f32),
        jax.ShapeDtypeStruct((E,), _f32),
        jax.ShapeDtypeStruct((E,), _f32),
    ),
    mesh=_sc_mesh(),
    compiler_params=pltpu.CompilerParams(needs_layout_passes=False),
    scratch_types=(
        pltpu.VMEM((CH,), jnp.int32),
        pltpu.VMEM((CH,), jnp.int32),
        pltpu.VMEM((CH, EHID), _f32),
        pltpu.VMEM((CH, EHID), _f32),
        pltpu.VMEM((N,), _f32),
        pltpu.VMEM((N,), _f32),
        pltpu.VMEM((CH,), _f32),
        pltpu.VMEM((CH,), _f32),
        pltpu.SemaphoreType.DMA,
        pltpu.SemaphoreType.DMA,
    ),
  )


BE = 512


def _edge_body(gsum_ref, e_ref, ag_ref, adg_ref, c_ref, w2_ref, b2_ref,
               lew_ref, aew_ref, en_ref, ex_ref, *, edim):
    g = gsum_ref[...]
    if edim == 1:
        pre = g + e_ref[...][:, None] * c_ref[...][None, :]
    else:
        pre = g + jnp.dot(e_ref[...], c_ref[...], preferred_element_type=_f32)
    pre = jnp.maximum(pre, 0.0)
    en = jnp.dot(pre, w2_ref[...], preferred_element_type=_f32) \
        + b2_ref[...][None, :]
    v = jnp.sum(lew_ref[...] * aew_ref[...][None, :], axis=1)
    aedge = jnp.sum(en * v[None, :], axis=1)
    alpha = ag_ref[...] + adg_ref[...] + aedge
    alpha = jnp.where(alpha > 0, alpha, 0.2 * alpha)
    en_ref[...] = en
    ex_ref[...] = jnp.exp(alpha)


def _make_edge(edim):
    if edim == 1:
        e_spec = pl.BlockSpec((BE,), lambda i: (i,))
        c_spec = pl.BlockSpec((EHID,), lambda i: (0,))
    else:
        e_spec = pl.BlockSpec((BE, edim), lambda i: (i, 0))
        c_spec = pl.BlockSpec((edim, EHID), lambda i: (0, 0))
    return pl.pallas_call(
        functools.partial(_edge_body, edim=edim),
        grid=(E // BE,),
        in_specs=[
            pl.BlockSpec((BE, EHID), lambda i: (i, 0)),
            e_spec,
            pl.BlockSpec((BE,), lambda i: (i,)),
            pl.BlockSpec((BE,), lambda i: (i,)),
            c_spec,
            pl.BlockSpec((EHID, EDIM), lambda i: (0, 0)),
            pl.BlockSpec((EDIM,), lambda i: (0,)),
            pl.BlockSpec((EDIM, D), lambda i: (0, 0)),
            pl.BlockSpec((D,), lambda i: (0,)),
        ],
        out_specs=[
            pl.BlockSpec((BE, EDIM), lambda i: (i, 0)),
            pl.BlockSpec((BE,), lambda i: (i,)),
        ],
        out_shape=(
            jax.ShapeDtypeStruct((E, EDIM), _f32),
            jax.ShapeDtypeStruct((E,), _f32),
        ),
        compiler_params=pltpu.CompilerParams(
            dimension_semantics=("arbitrary",)),
    )


_edge0 = _make_edge(1)
_edge16 = _make_edge(EDIM)


def _scagg_body(h_hbm, src_hbm, dst_hbm, ex_hbm, z2_hbm, z1_hbm,
                pp_hbm, dp_hbm,
                idx_s, idx_d, ex_v, rows, out_sh, den_sh, sem):
    cid = lax.axis_index("c")
    sid = lax.axis_index("s")
    wid = sid * NC + cid
    base = wid * EPW

    @pl.when(sid == 0)
    def _():
        pltpu.sync_copy(z2_hbm, out_sh)
        pltpu.sync_copy(z1_hbm, den_sh)
    plsc.subcore_barrier()

    def chunk(i, carry):
        off = base + i * CH
        pltpu.sync_copy(src_hbm.at[pl.ds(off, CH)], idx_s)
        pltpu.sync_copy(dst_hbm.at[pl.ds(off, CH)], idx_d)
        pltpu.sync_copy(ex_hbm.at[pl.ds(off, CH)], ex_v)
        pltpu.async_copy(h_hbm.at[idx_s], rows, sem).wait()

        def scale(g, c2):
            exv = ex_v[pl.ds(g * 16, 16)]
            for j in range(16):
                c = exv[j]
                e = g * 16 + j
                for k in range(D // 16):
                    sl = pl.ds(k * 16, 16)
                    rows[e, sl] = rows[e, sl] * c
            return c2
        lax.fori_loop(0, CH // 16, scale, 0)

        pltpu.sync_copy(rows, out_sh.at[idx_d], add=True)
        pltpu.sync_copy(ex_v, den_sh.at[idx_d], add=True)
        return carry
    lax.fori_loop(0, NCHUNK, chunk, 0)

    plsc.subcore_barrier()

    @pl.when(sid == 0)
    def _():
        pltpu.sync_copy(out_sh, pp_hbm.at[cid])
        pltpu.sync_copy(den_sh, dp_hbm.at[cid])


@functools.cache
def _scagg():
  return pl.kernel(
    _scagg_body,
    out_type=(
        jax.ShapeDtypeStruct((NC, N, D), _f32),
        jax.ShapeDtypeStruct((NC, N), _f32),
    ),
    mesh=_sc_mesh(),
    compiler_params=pltpu.CompilerParams(needs_layout_passes=False),
    scratch_types=(
        pltpu.VMEM((CH,), jnp.int32),
        pltpu.VMEM((CH,), jnp.int32),
        pltpu.VMEM((CH,), _f32),
        pltpu.VMEM((CH, D), _f32),
        pltpu.VMEM_SHARED((N, D), _f32),
        pltpu.VMEM_SHARED((N,), _f32),
        pltpu.SemaphoreType.DMA,
    ),
  )


def _pool_body(pp_ref, dp_ref, gb_ref, gam_ref, bet_ref, batch_ref,
               w1_ref, b1_ref, w2_ref, b2_ref, w3_ref, b3_ref, out_ref):
    x = _node_epilogue(pp_ref, dp_ref, gb_ref, gam_ref, bet_ref)
    b = batch_ref[...]
    oh = (b[:, None] == lax.broadcasted_iota(jnp.int32, (N, G), 1)) \
        .astype(_f32)
    sums = lax.dot_general(oh, x, (((0,), (0,)), ((), ())),
                           preferred_element_type=_f32)
    cnt = jnp.sum(oh, axis=0)
    pooled = sums / jnp.maximum(cnt, 1.0)[:, None]
    z = jnp.maximum(jnp.dot(pooled, w1_ref[...],
                            preferred_element_type=_f32)
                    + b1_ref[...][None, :], 0.0)
    z = jnp.maximum(jnp.dot(z, w2_ref[...], preferred_element_type=_f32)
                    + b2_ref[...][None, :], 0.0)
    out_ref[...] = jnp.dot(z, w3_ref[...], preferred_element_type=_f32) \
        + b3_ref[...][None, :]


_pool = pl.pallas_call(
    _pool_body,
    out_shape=jax.ShapeDtypeStruct((G, 1), _f32),
)


def kernel(x, edge_index, edge_attr, batch, params):
    src = edge_index[0]
    dst = edge_index[1]
    layers = params['layers']
    z2 = jnp.zeros((N, D), _f32)
    z1 = jnp.zeros((N,), _f32)

    e = jnp.reshape(edge_attr, (E,))
    pp = dp = None
    for i in range(3):
        p = layers[i]
        ein = 2 * D + (1 if i == 0 else EDIM)
        w1a = p['e_w1'][:D]
        w1b = p['e_w1'][D:2 * D]
        w1c = p['e_w1'][2 * D:]
        if i == 0:
            w1c = jnp.reshape(w1c, (EHID,))
            xa, xb, h, a_src, a_dst = _node0(
                x, w1a, w1b, p['e_b1'], p['lin_w'],
                p['att_src'], p['att_dst'])
        else:
            pprev = layers[i - 1]
            xa, xb, h, a_src, a_dst = _nodefin(
                pp, dp, pprev['gat_bias'], pprev['bn_gamma'],
                pprev['bn_beta'], w1a, w1b, p['e_b1'], p['lin_w'],
                p['att_src'], p['att_dst'])
        gsum, ag, adg = _scgather()(xa, xb, a_src, a_dst, src, dst)
        edge_fn = _edge0 if i == 0 else _edge16
        e, ex0 = edge_fn(gsum, e, ag, adg, w1c, p['e_w2'], p['e_b2'],
                         p['lin_edge_w'], p['att_edge'])
        pp, dp = _scagg()(h, src, dst, ex0, z2, z1)

    plast = layers[2]
    m = params['mlp']
    return _pool(pp, dp, plast['gat_bias'], plast['bn_gamma'],
                 plast['bn_beta'], batch, m['w1'], m['b1'],
                 m['w2'], m['b2'], m['w3'], m['b3'])
